# baseline (speedup 1.0000x reference)
import functools

import jax
import jax.numpy as jnp
from jax import lax
from jax.experimental import pallas as pl
from jax.experimental.pallas import tpu as pltpu
from jax.experimental.pallas import tpu_sc as plsc

_VOCAB = 1000000
_D = 64
_B_TOTAL = 4096 * 200
_NW = 32
_B_PER_W = _B_TOTAL // _NW
_CHUNK = 1024
_NCHUNK = _B_PER_W // _CHUNK

_mesh = plsc.VectorSubcoreMesh(core_axis_name="c", subcore_axis_name="s")


@functools.partial(
    pl.kernel,
    out_type=jax.ShapeDtypeStruct((_B_TOTAL, _D), jnp.float32),
    mesh=_mesh,
    scratch_types=[
        pltpu.VMEM((_CHUNK,), jnp.int32),
        pltpu.VMEM((_CHUNK, _D), jnp.float32),
        pltpu.SemaphoreType.DMA,
    ],
    compiler_params=pltpu.CompilerParams(use_tc_tiling_on_sc=False),
)
def _embed_gather(idx_hbm, table_hbm, out_hbm, idx_v, rows_v, sem):
    wid = lax.axis_index("s") * 2 + lax.axis_index("c")
    base = wid * _B_PER_W

    @pl.loop(0, _NCHUNK)
    def _chunk(i):
        off = base + i * _CHUNK
        pltpu.sync_copy(idx_hbm.at[pl.ds(off, _CHUNK)], idx_v)
        pltpu.async_copy(table_hbm.at[idx_v], rows_v, sem).wait()
        pltpu.sync_copy(rows_v, out_hbm.at[pl.ds(off, _CHUNK)])


def kernel(x, table):
    flat = x.reshape(-1)
    out = _embed_gather(flat, table)
    return out.reshape(x.shape + (table.shape[1],))


# speedup vs baseline: 1.0146x; 1.0146x over previous
import functools

import jax
import jax.numpy as jnp
from jax import lax
from jax.experimental import pallas as pl
from jax.experimental.pallas import tpu as pltpu
from jax.experimental.pallas import tpu_sc as plsc

_D = 64
_B_TOTAL = 4096 * 200
_NW = 32
_B_PER_W = _B_TOTAL // _NW
_CHUNK = 800
_NCHUNK = _B_PER_W // _CHUNK

_mesh = plsc.VectorSubcoreMesh(core_axis_name="c", subcore_axis_name="s")


@functools.partial(
    pl.kernel,
    out_type=jax.ShapeDtypeStruct((_B_TOTAL, _D), jnp.float32),
    mesh=_mesh,
    scratch_types=[
        pltpu.VMEM((_B_PER_W,), jnp.int32),
        pltpu.VMEM((_CHUNK, _D), jnp.float32),
        pltpu.VMEM((_CHUNK, _D), jnp.float32),
        pltpu.SemaphoreType.DMA,
        pltpu.SemaphoreType.DMA,
    ],
    compiler_params=pltpu.CompilerParams(use_tc_tiling_on_sc=False),
)
def _embed_gather(idx_hbm, table_hbm, out_hbm, idx_v, rows0, rows1, gsem, wsem):
    wid = lax.axis_index("s") * 2 + lax.axis_index("c")
    base = wid * _B_PER_W
    pltpu.sync_copy(idx_hbm.at[pl.ds(base, _B_PER_W)], idx_v)

    def gather_chunk(c, buf):
        pltpu.async_copy(
            table_hbm.at[idx_v.at[pl.ds(c * _CHUNK, _CHUNK)]], buf, gsem
        ).wait()

    def start_write(c, buf):
        pltpu.async_copy(buf, out_hbm.at[pl.ds(base + c * _CHUNK, _CHUNK)], wsem)

    def wait_write(c, buf):
        pltpu.make_async_copy(buf, out_hbm.at[pl.ds(base + c * _CHUNK, _CHUNK)], wsem).wait()

    gather_chunk(0, rows0)

    @pl.loop(0, _NCHUNK // 2)
    def _pair(k):
        c0 = 2 * k
        start_write(c0, rows0)
        gather_chunk(c0 + 1, rows1)
        wait_write(c0, rows0)
        start_write(c0 + 1, rows1)
        @pl.when(c0 + 2 < _NCHUNK)
        def _():
            gather_chunk(c0 + 2, rows0)
        wait_write(c0 + 1, rows1)


def kernel(x, table):
    flat = x.reshape(-1)
    out = _embed_gather(flat, table)
    return out.reshape(x.shape + (table.shape[1],))
